# baseline (speedup 1.0000x reference)
import functools

import jax
import jax.numpy as jnp
from jax import lax
from jax.experimental import pallas as pl
from jax.experimental.pallas import tpu as pltpu
from jax.experimental.pallas import tpu_sc as plsc

T, C = 2048, 768
NH, NKV, HD = 12, 4, 64
HHD = HD // 2
E_MLP, E_VE = 8, 2
E_TOT = E_MLP + E_VE
HID = 768
VOCAB = 32768

BT = 256
BQ = 256
BK = 256
BTM = 512

SC_NC, SC_NS = 2, 16
SC_NW = SC_NC * SC_NS
VE_ROWS = E_VE * T
VE_PER_W = VE_ROWS // SC_NW


def _rsqrt_norm(x):
    return x * lax.rsqrt(jnp.mean(x * x, axis=-1, keepdims=True) + 1e-6)


def _qkv_body(x_ref, cosq_ref, sinq_ref, rot_ref, seg_ref, segt_ref,
              wq_ref, wk_ref, wv_ref, q_ref, k_ref, v_ref):
    xn = _rsqrt_norm(x_ref[...])
    q = jnp.dot(xn, wq_ref[...], preferred_element_type=jnp.float32)
    k = jnp.dot(xn, wk_ref[...], preferred_element_type=jnp.float32)
    v = jnp.dot(xn, wv_ref[...], preferred_element_type=jnp.float32)

    def rope_norm_full(t, nh):
        w = nh * HD
        rot = jnp.dot(t, rot_ref[:w, :w], preferred_element_type=jnp.float32)
        r = t * cosq_ref[:, :w] + rot * sinq_ref[:, :w]
        sums = jnp.dot(r * r, seg_ref[:w, :nh],
                       preferred_element_type=jnp.float32)
        rstd = lax.rsqrt(sums * (1.0 / HD) + 1e-6)
        rstdb = jnp.dot(rstd, segt_ref[:nh, :w],
                        preferred_element_type=jnp.float32)
        return r * rstdb

    rep = NH // NKV
    nq = rope_norm_full(q, NH)
    nk = rope_norm_full(k, NKV)
    for g in range(NKV):
        for j in range(rep):
            h = g * rep + j
            q_ref[g, j] = nq[:, h * HD:(h + 1) * HD]
        k_ref[g] = nk[:, g * HD:(g + 1) * HD]
        v_ref[g] = jnp.concatenate(
            [v[:, g * HD:(g + 1) * HD],
             jnp.ones((v.shape[0], 1), jnp.float32),
             jnp.zeros((v.shape[0], 128 - HD - 1), jnp.float32)], axis=1)


def _qkv_call(xf, cos2, sin2, wq_t, wk_t, wv_t):
    cosq = jnp.tile(cos2, (1, (NH * HD) // HHD))
    sinq = jnp.tile(jnp.concatenate([sin2, -sin2], axis=1), (1, NH))
    ii = jnp.arange(NH * HD)
    rot = ((jnp.abs(ii[:, None] - ii[None, :]) == HHD)
           & (ii[:, None] // HD == ii[None, :] // HD)).astype(jnp.float32)
    seg = (ii[:, None] // HD == jnp.arange(16)[None, :]).astype(jnp.float32)
    segt = seg.T
    return pl.pallas_call(
        _qkv_body,
        grid=(T // BT,),
        in_specs=[
            pl.BlockSpec((BT, C), lambda i: (i, 0)),
            pl.BlockSpec((BT, NH * HD), lambda i: (i, 0)),
            pl.BlockSpec((BT, NH * HD), lambda i: (i, 0)),
            pl.BlockSpec((NH * HD, NH * HD), lambda i: (0, 0)),
            pl.BlockSpec((NH * HD, 16), lambda i: (0, 0)),
            pl.BlockSpec((16, NH * HD), lambda i: (0, 0)),
            pl.BlockSpec((C, NH * HD), lambda i: (0, 0)),
            pl.BlockSpec((C, NKV * HD), lambda i: (0, 0)),
            pl.BlockSpec((C, NKV * HD), lambda i: (0, 0)),
        ],
        out_specs=[
            pl.BlockSpec((NKV, NH // NKV, BT, HD), lambda i: (0, 0, i, 0)),
            pl.BlockSpec((NKV, BT, HD), lambda i: (0, i, 0)),
            pl.BlockSpec((NKV, BT, 128), lambda i: (0, i, 0)),
        ],
        out_shape=[
            jax.ShapeDtypeStruct((NKV, NH // NKV, T, HD), jnp.float32),
            jax.ShapeDtypeStruct((NKV, T, HD), jnp.float32),
            jax.ShapeDtypeStruct((NKV, T, 128), jnp.float32),
        ],
    )(xf, cosq, sinq, rot, seg, segt, wq_t, wk_t, wv_t)


def _attn_body(q_ref, k_ref, v_ref, y_ref):
    rep = NH // NKV
    t = pl.program_id(1)
    q = q_ref[0].reshape(rep * BQ, HD) * 0.125

    def body(j, acc):
        kj = k_ref[0, pl.ds(j * BK, BK), :]
        vj = v_ref[0, pl.ds(j * BK, BK), :]
        s = lax.dot_general(q, kj, (((1,), (1,)), ((), ())),
                            preferred_element_type=jnp.float32)
        p = jnp.exp(s)
        return acc + jnp.dot(p, vj, preferred_element_type=jnp.float32)

    acc = lax.fori_loop(0, t, body,
                        jnp.zeros((rep * BQ, 128), jnp.float32))
    kj = k_ref[0, pl.ds(t * BK, BK), :]
    vj = v_ref[0, pl.ds(t * BK, BK), :]
    s = lax.dot_general(q, kj, (((1,), (1,)), ((), ())),
                        preferred_element_type=jnp.float32)
    mask = ((lax.broadcasted_iota(jnp.int32, (rep * BQ, BK), 0) & (BQ - 1))
            >= lax.broadcasted_iota(jnp.int32, (rep * BQ, BK), 1))
    p = jnp.where(mask, jnp.exp(s), 0.0)
    acc = acc + jnp.dot(p, vj, preferred_element_type=jnp.float32)
    y = acc[:, :HD] / acc[:, HD:HD + 1]
    y_ref[0] = y.reshape(rep, BQ, HD)


def _attn_call(q, k, v):
    rep = NH // NKV
    return pl.pallas_call(
        _attn_body,
        grid=(NKV, T // BQ),
        in_specs=[
            pl.BlockSpec((1, rep, BQ, HD), lambda g, t: (g, 0, t, 0)),
            pl.BlockSpec((1, T, HD), lambda g, t: (g, 0, 0)),
            pl.BlockSpec((1, T, 128), lambda g, t: (g, 0, 0)),
        ],
        out_specs=pl.BlockSpec((1, rep, BQ, HD), lambda g, t: (g, 0, t, 0)),
        out_shape=jax.ShapeDtypeStruct((NKV, rep, T, HD), jnp.float32),
    )(q, k, v)


def _post_body(x_ref, y_ref, wo_ref, rtr_ref,
               x2_ref, xn2_ref, rw_ref, wmap_ref):
    x2 = x_ref[...] + jnp.dot(y_ref[...], wo_ref[...],
                              preferred_element_type=jnp.float32)
    x2_ref[...] = x2
    xn2 = _rsqrt_norm(x2)
    xn2_ref[...] = xn2
    scores = jnp.dot(xn2, rtr_ref[...], preferred_element_type=jnp.float32)
    col = lax.broadcasted_iota(jnp.int32, scores.shape, 1)
    scores = jnp.where(col < E_TOT, scores, -1e30)
    mx = jnp.max(scores, axis=1, keepdims=True)
    ex = jnp.exp(scores - mx)
    rw = ex / jnp.sum(ex, axis=1, keepdims=True)
    rw_ref[...] = rw
    m1 = jnp.max(rw, axis=1, keepdims=True)
    idx1 = jnp.min(jnp.where(rw == m1, col, E_TOT + 7), axis=1, keepdims=True)
    rmask = jnp.where(col == idx1, -1.0, rw)
    m2 = jnp.max(rmask, axis=1, keepdims=True)
    idx2 = jnp.min(jnp.where(rmask == m2, col, E_TOT + 7), axis=1, keepdims=True)
    s = m1 + m2 + 1e-10
    wmap_ref[...] = (jnp.where(col == idx1, m1 / s, 0.0)
                     + jnp.where(col == idx2, m2 / s, 0.0))


def _post_call(xf, y, wo_t, rtr_pad):
    return pl.pallas_call(
        _post_body,
        grid=(T // BT,),
        in_specs=[
            pl.BlockSpec((BT, C), lambda i: (i, 0)),
            pl.BlockSpec((BT, C), lambda i: (i, 0)),
            pl.BlockSpec((C, C), lambda i: (0, 0)),
            pl.BlockSpec((C, 16), lambda i: (0, 0)),
        ],
        out_specs=[
            pl.BlockSpec((BT, C), lambda i: (i, 0)),
            pl.BlockSpec((BT, C), lambda i: (i, 0)),
            pl.BlockSpec((BT, 16), lambda i: (i, 0)),
            pl.BlockSpec((BT, 16), lambda i: (i, 0)),
        ],
        out_shape=[
            jax.ShapeDtypeStruct((T, C), jnp.float32),
            jax.ShapeDtypeStruct((T, C), jnp.float32),
            jax.ShapeDtypeStruct((T, 16), jnp.float32),
            jax.ShapeDtypeStruct((T, 16), jnp.float32),
        ],
    )(xf, y, wo_t, rtr_pad)


def _ve_gather(tables, ids):
    mesh = plsc.VectorSubcoreMesh(core_axis_name="c", subcore_axis_name="s")

    @functools.partial(
        pl.kernel, mesh=mesh,
        out_type=jax.ShapeDtypeStruct((VE_ROWS, C), jnp.float32),
        scratch_types=[
            pltpu.VMEM((VE_PER_W,), jnp.int32),
            pltpu.VMEM((VE_PER_W, C), jnp.float32),
            pltpu.SemaphoreType.DMA,
        ],
    )
    def k(tab_hbm, ids_hbm, out_hbm, idx_v, rows_v, sem):
        wid = lax.axis_index("s") * SC_NC + lax.axis_index("c")
        half = wid // (SC_NW // E_VE)
        idx_base = (wid - half * (SC_NW // E_VE)) * VE_PER_W
        pltpu.sync_copy(ids_hbm.at[pl.ds(idx_base, VE_PER_W)], idx_v)
        offset = half * VOCAB
        for i in range(VE_PER_W // 16):
            sl = pl.ds(i * 16, 16)
            idx_v[sl] = idx_v[sl] + offset
        pltpu.async_copy(tab_hbm.at[idx_v], rows_v, sem).wait()
        pltpu.sync_copy(rows_v, out_hbm.at[pl.ds(wid * VE_PER_W, VE_PER_W)])

    return k(tables, ids)


def _moe_body(x2_ref, xn2_ref, wmap_ref, ve0_ref, ve1_ref, fc_ref, proj_ref,
              out_ref):
    wmap = wmap_ref[...]
    col = lax.broadcasted_iota(jnp.int32, wmap.shape, 1)
    xb = xn2_ref[...].astype(jnp.bfloat16)
    h_all = lax.dot_general(xb, fc_ref[...], (((1,), (1,)), ((), ())),
                            preferred_element_type=jnp.float32)
    acc = None
    for e in range(E_MLP):
        we = jnp.sum(jnp.where(col == e, wmap, 0.0), axis=1, keepdims=True)
        he = h_all[:, e * HID:(e + 1) * HID]
        hw = (jnp.square(jnp.maximum(he, 0.0)) * we).astype(jnp.bfloat16)
        mo = lax.dot_general(hw, proj_ref[e], (((1,), (1,)), ((), ())),
                             preferred_element_type=jnp.float32)
        acc = mo if acc is None else acc + mo
    w8 = jnp.sum(jnp.where(col == E_MLP, wmap, 0.0), axis=1, keepdims=True)
    w9 = jnp.sum(jnp.where(col == E_MLP + 1, wmap, 0.0), axis=1,
                 keepdims=True)
    out_ref[...] = (x2_ref[...] + ve0_ref[...] * w8 + ve1_ref[...] * w9
                    + acc)


def _moe_call(x2, xn2, wmap, ve0, ve1, fc_w, proj_w):
    return pl.pallas_call(
        _moe_body,
        grid=(T // BTM,),
        in_specs=[
            pl.BlockSpec((BTM, C), lambda t: (t, 0)),
            pl.BlockSpec((BTM, C), lambda t: (t, 0)),
            pl.BlockSpec((BTM, 16), lambda t: (t, 0)),
            pl.BlockSpec((BTM, C), lambda t: (t, 0)),
            pl.BlockSpec((BTM, C), lambda t: (t, 0)),
            pl.BlockSpec((E_MLP * HID, C), lambda t: (0, 0)),
            pl.BlockSpec((E_MLP, C, HID), lambda t: (0, 0, 0)),
        ],
        out_specs=pl.BlockSpec((BTM, C), lambda t: (t, 0)),
        out_shape=jax.ShapeDtypeStruct((T, C), jnp.float32),
    )(x2, xn2, wmap, ve0, ve1,
      fc_w.astype(jnp.bfloat16).reshape(E_MLP * HID, C),
      proj_w.astype(jnp.bfloat16))


def kernel(x, cos, sin, token_ids, wq, wk, wv, wo, router_w, fc_w, proj_w,
           ve_tables):
    xf = x.reshape(T, C)
    cos2 = cos.reshape(T, HHD)
    sin2 = sin.reshape(T, HHD)
    ids = token_ids.reshape(T).astype(jnp.int32)
    tables = ve_tables.reshape(E_VE * VOCAB, C)
    rtr_pad = jnp.pad(router_w, ((0, 16 - E_TOT), (0, 0))).T

    ve = _ve_gather(tables, ids)
    q, k, v = _qkv_call(xf, cos2, sin2, wq.T, wk.T, wv.T)
    y3 = _attn_call(q, k, v)
    y = y3.reshape(NH, T, HD).transpose(1, 0, 2).reshape(T, C)
    x2, xn2, rw, wmap = _post_call(xf, y, wo.T, rtr_pad)
    out = _moe_call(x2, xn2, wmap, ve[:T], ve[T:], fc_w, proj_w)
    return out.reshape(1, T, C), rw[:, :E_TOT].reshape(1, T, E_TOT)


# speedup vs baseline: 1.3620x; 1.3620x over previous
import functools

import jax
import jax.numpy as jnp
from jax import lax
from jax.experimental import pallas as pl
from jax.experimental.pallas import tpu as pltpu
from jax.experimental.pallas import tpu_sc as plsc

T, C = 2048, 768
NH, NKV, HD = 12, 4, 64
HHD = HD // 2
E_MLP, E_VE = 8, 2
E_TOT = E_MLP + E_VE
HID = 768
VOCAB = 32768

BT = 256
BQ = 512
BK = 512
BTM = 512

SC_NC, SC_NS = 2, 16
SC_NW = SC_NC * SC_NS
VE_ROWS = E_VE * T
VE_PER_W = VE_ROWS // SC_NW


def _rsqrt_norm(x):
    return x * lax.rsqrt(jnp.mean(x * x, axis=-1, keepdims=True) + 1e-6)


def _qkv_body(x_ref, cos_ref, sin_ref, t32_ref, ts_ref, rot_ref, seg_ref,
              segt_ref, wq_ref, wk_ref, wv_ref, q_ref, k_ref, v_ref):
    xn = _rsqrt_norm(x_ref[...])
    q = jnp.dot(xn, wq_ref[...], preferred_element_type=jnp.float32)
    k = jnp.dot(xn, wk_ref[...], preferred_element_type=jnp.float32)
    v = jnp.dot(xn, wv_ref[...], preferred_element_type=jnp.float32)
    cosw = jnp.dot(cos_ref[...], t32_ref[...],
                   preferred_element_type=jnp.float32)
    sinw = jnp.dot(sin_ref[...], ts_ref[...],
                   preferred_element_type=jnp.float32)

    def rope_norm_full(t, nh):
        w = nh * HD
        rot = jnp.dot(t, rot_ref[:w, :w], preferred_element_type=jnp.float32)
        r = t * cosw[:, :w] + rot * sinw[:, :w]
        sums = jnp.dot(r * r, seg_ref[:w, :nh],
                       preferred_element_type=jnp.float32)
        rstd = lax.rsqrt(sums * (1.0 / HD) + 1e-6)
        rstdb = jnp.dot(rstd, segt_ref[:nh, :w],
                        preferred_element_type=jnp.float32)
        return r * rstdb

    rep = NH // NKV
    nq = rope_norm_full(q, NH)
    nk = rope_norm_full(k, NKV)
    for g in range(NKV):
        for j in range(rep):
            h = g * rep + j
            q_ref[g, j] = nq[:, h * HD:(h + 1) * HD]
        k_ref[g] = nk[:, g * HD:(g + 1) * HD]
        v_ref[g] = jnp.concatenate(
            [v[:, g * HD:(g + 1) * HD],
             jnp.ones((v.shape[0], 1), jnp.float32),
             jnp.zeros((v.shape[0], 128 - HD - 1), jnp.float32)], axis=1)


def _qkv_call(xf, cos2, sin2, wq_t, wk_t, wv_t):
    ii = jnp.arange(NH * HD)
    jm = ii % HD
    rot = ((jnp.abs(ii[:, None] - ii[None, :]) == HHD)
           & (ii[:, None] // HD == ii[None, :] // HD)).astype(jnp.float32)
    seg = (ii[:, None] // HD == jnp.arange(16)[None, :]).astype(jnp.float32)
    segt = seg.T
    i32 = jnp.arange(HHD)
    t32 = (i32[:, None] == (jm % HHD)[None, :]).astype(jnp.float32)
    ts = t32 * jnp.where(jm < HHD, 1.0, -1.0)[None, :]
    return pl.pallas_call(
        _qkv_body,
        grid=(T // BT,),
        in_specs=[
            pl.BlockSpec((BT, C), lambda i: (i, 0)),
            pl.BlockSpec((BT, HHD), lambda i: (i, 0)),
            pl.BlockSpec((BT, HHD), lambda i: (i, 0)),
            pl.BlockSpec((HHD, NH * HD), lambda i: (0, 0)),
            pl.BlockSpec((HHD, NH * HD), lambda i: (0, 0)),
            pl.BlockSpec((NH * HD, NH * HD), lambda i: (0, 0)),
            pl.BlockSpec((NH * HD, 16), lambda i: (0, 0)),
            pl.BlockSpec((16, NH * HD), lambda i: (0, 0)),
            pl.BlockSpec((C, NH * HD), lambda i: (0, 0)),
            pl.BlockSpec((C, NKV * HD), lambda i: (0, 0)),
            pl.BlockSpec((C, NKV * HD), lambda i: (0, 0)),
        ],
        out_specs=[
            pl.BlockSpec((NKV, NH // NKV, BT, HD), lambda i: (0, 0, i, 0)),
            pl.BlockSpec((NKV, BT, HD), lambda i: (0, i, 0)),
            pl.BlockSpec((NKV, BT, 128), lambda i: (0, i, 0)),
        ],
        out_shape=[
            jax.ShapeDtypeStruct((NKV, NH // NKV, T, HD), jnp.float32),
            jax.ShapeDtypeStruct((NKV, T, HD), jnp.float32),
            jax.ShapeDtypeStruct((NKV, T, 128), jnp.float32),
        ],
    )(xf, cos2, sin2, t32, ts, rot, seg, segt, wq_t, wk_t, wv_t)


def _attn_body(q_ref, k_ref, v_ref, y_ref):
    rep = NH // NKV
    t = pl.program_id(1)
    q = q_ref[0].reshape(rep * BQ, HD) * 0.125

    def body(j, acc):
        kj = k_ref[0, pl.ds(j * BK, BK), :]
        vj = v_ref[0, pl.ds(j * BK, BK), :]
        s = lax.dot_general(q, kj, (((1,), (1,)), ((), ())),
                            preferred_element_type=jnp.float32)
        p = jnp.exp(s)
        return acc + jnp.dot(p, vj, preferred_element_type=jnp.float32)

    acc = lax.fori_loop(0, t, body, jnp.zeros((rep * BQ, 128), jnp.float32))
    kj = k_ref[0, pl.ds(t * BK, BK), :]
    vj = v_ref[0, pl.ds(t * BK, BK), :]
    s = lax.dot_general(q, kj, (((1,), (1,)), ((), ())),
                        preferred_element_type=jnp.float32)
    mask = ((lax.broadcasted_iota(jnp.int32, (rep * BQ, BK), 0) & (BQ - 1))
            >= lax.broadcasted_iota(jnp.int32, (rep * BQ, BK), 1))
    p = jnp.where(mask, jnp.exp(s), 0.0)
    acc = acc + jnp.dot(p, vj, preferred_element_type=jnp.float32)
    y = acc[:, :HD] / acc[:, HD:HD + 1]
    y_ref[0] = y.reshape(rep, BQ, HD)


def _attn_call(q, k, v):
    rep = NH // NKV
    return pl.pallas_call(
        _attn_body,
        grid=(NKV, T // BQ),
        in_specs=[
            pl.BlockSpec((1, rep, BQ, HD), lambda g, t: (g, 0, t, 0)),
            pl.BlockSpec((1, T, HD), lambda g, t: (g, 0, 0)),
            pl.BlockSpec((1, T, 128), lambda g, t: (g, 0, 0)),
        ],
        out_specs=pl.BlockSpec((1, rep, BQ, HD), lambda g, t: (g, 0, t, 0)),
        out_shape=jax.ShapeDtypeStruct((NKV, rep, T, HD), jnp.float32),
    )(q, k, v)


def _pm_body(x_ref, y_ref, wo_ref, rtr_ref, ve0_ref, ve1_ref, fc_ref,
             proj_ref, out_ref, rw_ref):
    x2 = x_ref[...] + jnp.dot(y_ref[...], wo_ref[...],
                              preferred_element_type=jnp.float32)
    xn2 = _rsqrt_norm(x2)
    scores = jnp.dot(xn2, rtr_ref[...], preferred_element_type=jnp.float32)
    col = lax.broadcasted_iota(jnp.int32, scores.shape, 1)
    scores = jnp.where(col < E_TOT, scores, -1e30)
    mx = jnp.max(scores, axis=1, keepdims=True)
    ex = jnp.exp(scores - mx)
    rw = ex / jnp.sum(ex, axis=1, keepdims=True)
    rw_ref[...] = rw
    m1 = jnp.max(rw, axis=1, keepdims=True)
    idx1 = jnp.min(jnp.where(rw == m1, col, E_TOT + 7), axis=1, keepdims=True)
    rmask = jnp.where(col == idx1, -1.0, rw)
    m2 = jnp.max(rmask, axis=1, keepdims=True)
    idx2 = jnp.min(jnp.where(rmask == m2, col, E_TOT + 7), axis=1,
                   keepdims=True)
    sn = m1 + m2 + 1e-10
    wmap = (jnp.where(col == idx1, m1 / sn, 0.0)
            + jnp.where(col == idx2, m2 / sn, 0.0))
    xb = xn2.astype(jnp.bfloat16)
    h_all = lax.dot_general(xb, fc_ref[...], (((1,), (1,)), ((), ())),
                            preferred_element_type=jnp.float32)
    acc = None
    for e in range(E_MLP):
        we = jnp.sum(jnp.where(col == e, wmap, 0.0), axis=1, keepdims=True)
        he = h_all[:, e * HID:(e + 1) * HID]
        hw = (jnp.square(jnp.maximum(he, 0.0)) * we).astype(jnp.bfloat16)
        mo = lax.dot_general(hw, proj_ref[e], (((1,), (1,)), ((), ())),
                             preferred_element_type=jnp.float32)
        acc = mo if acc is None else acc + mo
    w8 = jnp.sum(jnp.where(col == E_MLP, wmap, 0.0), axis=1, keepdims=True)
    w9 = jnp.sum(jnp.where(col == E_MLP + 1, wmap, 0.0), axis=1,
                 keepdims=True)
    out_ref[...] = x2 + ve0_ref[...] * w8 + ve1_ref[...] * w9 + acc


def _pm_call(xf, y, wo_t, rtr_pad, ve0, ve1, fc_w, proj_w):
    return pl.pallas_call(
        _pm_body,
        grid=(T // BTM,),
        in_specs=[
            pl.BlockSpec((BTM, C), lambda t: (t, 0)),
            pl.BlockSpec((BTM, C), lambda t: (t, 0)),
            pl.BlockSpec((C, C), lambda t: (0, 0)),
            pl.BlockSpec((C, 16), lambda t: (0, 0)),
            pl.BlockSpec((BTM, C), lambda t: (t, 0)),
            pl.BlockSpec((BTM, C), lambda t: (t, 0)),
            pl.BlockSpec((E_MLP * HID, C), lambda t: (0, 0)),
            pl.BlockSpec((E_MLP, C, HID), lambda t: (0, 0, 0)),
        ],
        out_specs=[
            pl.BlockSpec((BTM, C), lambda t: (t, 0)),
            pl.BlockSpec((BTM, 16), lambda t: (t, 0)),
        ],
        out_shape=[
            jax.ShapeDtypeStruct((T, C), jnp.float32),
            jax.ShapeDtypeStruct((T, 16), jnp.float32),
        ],
    )(xf, y, wo_t, rtr_pad, ve0, ve1,
      fc_w.astype(jnp.bfloat16).reshape(E_MLP * HID, C),
      proj_w.astype(jnp.bfloat16))


def _ve_gather(tables, ids):
    mesh = plsc.VectorSubcoreMesh(core_axis_name="c", subcore_axis_name="s")

    @functools.partial(
        pl.kernel, mesh=mesh,
        out_type=[jax.ShapeDtypeStruct((T, C), jnp.float32),
                  jax.ShapeDtypeStruct((T, C), jnp.float32)],
        scratch_types=[
            pltpu.VMEM((VE_PER_W,), jnp.int32),
            pltpu.VMEM((VE_PER_W, C), jnp.float32),
            pltpu.SemaphoreType.DMA,
        ],
    )
    def k(tab_hbm, ids_hbm, out0_hbm, out1_hbm, idx_v, rows_v, sem):
        wid = lax.axis_index("s") * SC_NC + lax.axis_index("c")
        half = wid // (SC_NW // E_VE)
        base = (wid - half * (SC_NW // E_VE)) * VE_PER_W
        pltpu.sync_copy(ids_hbm.at[pl.ds(base, VE_PER_W)], idx_v)
        offset = half * VOCAB
        for i in range(VE_PER_W // 16):
            sl = pl.ds(i * 16, 16)
            idx_v[sl] = idx_v[sl] + offset
        pltpu.async_copy(tab_hbm.at[idx_v], rows_v, sem).wait()

        @pl.when(half == 0)
        def _():
            pltpu.sync_copy(rows_v, out0_hbm.at[pl.ds(base, VE_PER_W)])

        @pl.when(half == 1)
        def _():
            pltpu.sync_copy(rows_v, out1_hbm.at[pl.ds(base, VE_PER_W)])

    return k(tables, ids)


def kernel(x, cos, sin, token_ids, wq, wk, wv, wo, router_w, fc_w, proj_w,
           ve_tables):
    xf = x.reshape(T, C)
    cos2 = cos.reshape(T, HHD)
    sin2 = sin.reshape(T, HHD)
    ids = token_ids.reshape(T).astype(jnp.int32)
    tables = ve_tables.reshape(E_VE * VOCAB, C)
    rtr_pad = jnp.pad(router_w, ((0, 16 - E_TOT), (0, 0))).T

    ve0, ve1 = _ve_gather(tables, ids)
    q, k, v = _qkv_call(xf, cos2, sin2, wq.T, wk.T, wv.T)
    y3 = _attn_call(q, k, v)
    y = y3.reshape(NH, T, HD).transpose(1, 0, 2).reshape(T, C)
    out, rw = _pm_call(xf, y, wo.T, rtr_pad, ve0, ve1, fc_w, proj_w)
    return out.reshape(1, T, C), rw[:, :E_TOT].reshape(1, T, E_TOT)
